# Initial kernel scaffold; baseline (speedup 1.0000x reference)
import jax
import jax.numpy as jnp
from jax.experimental import pallas as pl


def kernel(infeat, labelTpesudo, labelT, FeatureDA, k):
    raise NotImplementedError("write your pallas kernel here")


# speedup vs baseline: 5.7489x; 5.7489x over previous
import functools

import jax
import jax.numpy as jnp
from jax import lax
from jax.experimental import pallas as pl
from jax.experimental.pallas import tpu as pltpu
from jax.experimental.pallas import tpu_sc as plsc

_ORISIZE = 256
_KER = 32
_P = _ORISIZE - _KER + 1
_HALF = _KER // 2
_B = 4
_CFEAT = 256
_NPATCH = 16

_NC, _NS = 2, 16


def _score_kernel(infeat_ref, prov_ref, pxy_ref, coords_ref):
    iy = lax.broadcasted_iota(jnp.int32, (_P, _P), 0)
    ix = lax.broadcasted_iota(jnp.int32, (_P, _P), 1)
    flat = iy * _P + ix
    big = jnp.int32(1 << 30)
    for b in range(_B):
        x0 = infeat_ref[b, 0]
        x1 = infeat_ref[b, 1]
        m = jnp.maximum(x0, x1)
        e0 = jnp.exp(x0 - m)
        e1 = jnp.exp(x1 - m)
        den = e0 + e1
        for c in range(2):
            sm = (e0 if c == 0 else e1) / den
            s = sm
            for d in (1, 2, 4, 8, 16):
                s = s + jnp.concatenate(
                    [s[:, d:], jnp.zeros((_ORISIZE, d), s.dtype)], axis=1)
            for d in (1, 2, 4, 8, 16):
                s = s + jnp.concatenate(
                    [s[d:, :], jnp.zeros((d, _ORISIZE), s.dtype)], axis=0)
            pooled = s[:_P, :_P] * (1.0 / (_KER * _KER))

            m1 = jnp.max(pooled)
            idx1 = jnp.min(jnp.where(pooled == m1, flat, big))
            py1 = idx1 // _P
            px1 = idx1 % _P
            oy0 = jnp.maximum(0, py1 - _HALF)
            oy1 = jnp.minimum(_P, py1 + _HALF)
            ox0 = jnp.maximum(0, px1 - _HALF)
            ox1 = jnp.minimum(_P, px1 + _HALF)
            region = (iy >= oy0) & (iy < oy1) & (ix >= ox0) & (ix < ox1)
            filt2 = jnp.where(region, jnp.float32(0.0), pooled)
            m2 = jnp.max(filt2)
            idx2 = jnp.min(jnp.where(filt2 == m2, flat, big))
            py2 = idx2 // _P
            px2 = idx2 % _P

            for kk, (mv, py, px) in enumerate(((m1, py1, px1),
                                               (m2, py2, px2))):
                p = c * (2 * _B) + kk * _B + b
                prov_ref[p] = mv
                pxy_ref[p, 0, 0] = px
                pxy_ref[p, 0, 1] = px + (_KER - 1)
                pxy_ref[p, 1, 0] = py
                pxy_ref[p, 1, 1] = py + (_KER - 1)
                coords_ref[p, 0] = b
                coords_ref[p, 1] = py
                coords_ref[p, 2] = px
                for f in range(3, 16):
                    coords_ref[p, f] = 0


_score_call = pl.pallas_call(
    _score_kernel,
    out_shape=(
        jax.ShapeDtypeStruct((_NPATCH,), jnp.float32),
        jax.ShapeDtypeStruct((_NPATCH, 2, 2), jnp.int32),
        jax.ShapeDtypeStruct((_NPATCH, 16), jnp.int32),
    ),
    in_specs=[pl.BlockSpec(memory_space=pltpu.VMEM)],
    out_specs=(
        pl.BlockSpec(memory_space=pltpu.SMEM),
        pl.BlockSpec(memory_space=pltpu.SMEM),
        pl.BlockSpec(memory_space=pltpu.SMEM),
    ),
)


_W = 40
_CCH = 32
_NCHUNK = _CFEAT // _CCH
_LANES = 16


def _patch_coords(coords_h, cvec, p):
    pltpu.sync_copy(coords_h.at[p], cvec)
    vec = cvec[...]
    lanes = lax.iota(jnp.int32, _LANES)
    zero = jnp.zeros((_LANES,), jnp.int32)
    b = jnp.max(jnp.where(lanes == 0, vec, zero))
    py = jnp.max(jnp.where(lanes == 1, vec, zero))
    px = jnp.max(jnp.where(lanes == 2, vec, zero))
    px8 = jnp.minimum((px // 8) * 8, _ORISIZE - _W)
    q = px - px8
    return b, py, px, px8, q


def _realign(gbuf, obuf, nch, q):
    lanes = lax.iota(jnp.int32, _LANES)

    def body(ch, carry):
        chv = jnp.full((_LANES,), ch, jnp.int32)
        for r in range(_KER):
            rv = jnp.full((_LANES,), r, jnp.int32)
            for h in (0, _LANES):
                v = plsc.load_gather(gbuf, [chv, rv, q + h + lanes])
                plsc.store_scatter(obuf, [chv, rv, h + lanes], v)
        return carry

    lax.fori_loop(0, nch, body, 0)


def _sc_gather(infeat_h, lp_h, lt_h, fda_h, coords_h,
               cls_o, feat_o, pse_o, lab_o,
               cvec, gbuf, obuf, gs2, os2, gs1, os1):
    w = lax.axis_index("s") * _NC + lax.axis_index("c")

    for j in range(4):
        t = w + 32 * j
        p = t // _NCHUNK
        chunk = t % _NCHUNK
        b, py, px, px8, q = _patch_coords(coords_h, cvec, p)
        pltpu.sync_copy(
            fda_h.at[b, pl.ds(chunk * _CCH, _CCH),
                     pl.ds(py, _KER), pl.ds(px8, _W)],
            gbuf)
        _realign(gbuf, obuf, _CCH, q)
        pltpu.sync_copy(obuf, feat_o.at[p, pl.ds(chunk * _CCH, _CCH)])

    @pl.when(w >= 16)
    def _():
        p = w - 16
        b, py, px, px8, q = _patch_coords(coords_h, cvec, p)
        pltpu.sync_copy(
            infeat_h.at[b, :, pl.ds(py, _KER), pl.ds(px8, _W)], gs2)
        _realign(gs2, os2, 2, q)
        pltpu.sync_copy(os2, cls_o.at[p])
        for src_h, dst_o in ((lp_h, pse_o), (lt_h, lab_o)):
            pltpu.sync_copy(
                src_h.at[b, :, pl.ds(py, _KER), pl.ds(px8, _W)], gs1)
            _realign(gs1, os1, 1, q)
            pltpu.sync_copy(os1, dst_o.at[p])


@functools.lru_cache(maxsize=1)
def _make_gather_call():
    return functools.partial(
        pl.kernel,
        out_type=(
            jax.ShapeDtypeStruct((_NPATCH, 2, _KER, _KER), jnp.float32),
            jax.ShapeDtypeStruct((_NPATCH, _CFEAT, _KER, _KER), jnp.float32),
            jax.ShapeDtypeStruct((_NPATCH, 1, _KER, _KER), jnp.float32),
            jax.ShapeDtypeStruct((_NPATCH, 1, _KER, _KER), jnp.float32),
        ),
        mesh=plsc.VectorSubcoreMesh(core_axis_name="c", subcore_axis_name="s"),
        scratch_types=[
            pltpu.VMEM((_LANES,), jnp.int32),
            pltpu.VMEM((_CCH, _KER, _W), jnp.float32),
            pltpu.VMEM((_CCH, _KER, _KER), jnp.float32),
            pltpu.VMEM((2, _KER, _W), jnp.float32),
            pltpu.VMEM((2, _KER, _KER), jnp.float32),
            pltpu.VMEM((1, _KER, _W), jnp.float32),
            pltpu.VMEM((1, _KER, _KER), jnp.float32),
        ],
        compiler_params=pltpu.CompilerParams(
            use_tc_tiling_on_sc=False, needs_layout_passes=False),
    )(_sc_gather)


def kernel(infeat, labelTpesudo, labelT, FeatureDA, k):
    del k
    prov, pxy, coords = _score_call(infeat)
    cls, feat, pse, lab = _make_gather_call()(
        infeat, labelTpesudo, labelT, FeatureDA, coords)
    return (cls, feat, pse, lab, prov, pxy)


# speedup vs baseline: 9.4191x; 1.6384x over previous
import functools

import jax
import jax.numpy as jnp
from jax import lax
from jax.experimental import pallas as pl
from jax.experimental.pallas import tpu as pltpu
from jax.experimental.pallas import tpu_sc as plsc

_ORISIZE = 256
_KER = 32
_P = _ORISIZE - _KER + 1
_HALF = _KER // 2
_B = 4
_CFEAT = 256
_NPATCH = 16

_NC, _NS = 2, 16


def _score_kernel(infeat_ref, prov_ref, pxy_ref, coords_ref):
    iy = lax.broadcasted_iota(jnp.int32, (_P, _P), 0)
    ix = lax.broadcasted_iota(jnp.int32, (_P, _P), 1)
    flat = iy * _P + ix
    big = jnp.int32(1 << 30)
    for b in range(_B):
        x0 = infeat_ref[b, 0]
        x1 = infeat_ref[b, 1]
        m = jnp.maximum(x0, x1)
        e0 = jnp.exp(x0 - m)
        e1 = jnp.exp(x1 - m)
        den = e0 + e1
        for c in range(2):
            sm = (e0 if c == 0 else e1) / den
            s = sm
            for d in (1, 2, 4, 8, 16):
                s = s + jnp.concatenate(
                    [s[:, d:], jnp.zeros((_ORISIZE, d), s.dtype)], axis=1)
            for d in (1, 2, 4, 8, 16):
                s = s + jnp.concatenate(
                    [s[d:, :], jnp.zeros((d, _ORISIZE), s.dtype)], axis=0)
            pooled = s[:_P, :_P] * (1.0 / (_KER * _KER))

            m1 = jnp.max(pooled)
            idx1 = jnp.min(jnp.where(pooled == m1, flat, big))
            py1 = idx1 // _P
            px1 = idx1 % _P
            oy0 = jnp.maximum(0, py1 - _HALF)
            oy1 = jnp.minimum(_P, py1 + _HALF)
            ox0 = jnp.maximum(0, px1 - _HALF)
            ox1 = jnp.minimum(_P, px1 + _HALF)
            region = (iy >= oy0) & (iy < oy1) & (ix >= ox0) & (ix < ox1)
            filt2 = jnp.where(region, jnp.float32(0.0), pooled)
            m2 = jnp.max(filt2)
            idx2 = jnp.min(jnp.where(filt2 == m2, flat, big))
            py2 = idx2 // _P
            px2 = idx2 % _P

            for kk, (mv, py, px) in enumerate(((m1, py1, px1),
                                               (m2, py2, px2))):
                p = c * (2 * _B) + kk * _B + b
                prov_ref[p] = mv
                pxy_ref[p, 0, 0] = px
                pxy_ref[p, 0, 1] = px + (_KER - 1)
                pxy_ref[p, 1, 0] = py
                pxy_ref[p, 1, 1] = py + (_KER - 1)
                coords_ref[p, 0, 0] = b
                coords_ref[p, 0, 1] = py
                coords_ref[p, 0, 2] = px
                for f in range(3, 16):
                    coords_ref[p, 0, f] = 0


_score_call = pl.pallas_call(
    _score_kernel,
    out_shape=(
        jax.ShapeDtypeStruct((_NPATCH,), jnp.float32),
        jax.ShapeDtypeStruct((_NPATCH, 2, 2), jnp.int32),
        jax.ShapeDtypeStruct((_NPATCH, 1, 16), jnp.int32),
    ),
    in_specs=[pl.BlockSpec(memory_space=pltpu.VMEM)],
    out_specs=(
        pl.BlockSpec(memory_space=pltpu.SMEM),
        pl.BlockSpec(memory_space=pltpu.SMEM),
        pl.BlockSpec(memory_space=pltpu.SMEM),
    ),
)


_NROW = 40
_CCH = 8
_NCHUNK = _CFEAT // _CCH
_LANES = 16


def _patch_coords(coords_h, cvec, p):
    pltpu.sync_copy(coords_h.at[p], cvec)
    vec = cvec[0]
    lanes = lax.iota(jnp.int32, _LANES)
    zero = jnp.zeros((_LANES,), jnp.int32)
    b = jnp.max(jnp.where(lanes == 0, vec, zero))
    py = jnp.max(jnp.where(lanes == 1, vec, zero))
    px = jnp.max(jnp.where(lanes == 2, vec, zero))
    py8 = jnp.minimum((py // 8) * 8, _ORISIZE - _NROW)
    qy = py - py8
    return b, py, px, py8, qy


def _realign(gbuf, obuf, nch, qy, px):
    lanes = lax.iota(jnp.int32, _LANES)

    def body(ch, carry):
        chv = jnp.full((_LANES,), ch, jnp.int32)
        for r in range(_KER):
            rv = jnp.full((_LANES,), qy + r, jnp.int32)
            dv = jnp.full((_LANES,), r, jnp.int32)
            for h in (0, _LANES):
                v = plsc.load_gather(gbuf, [chv, rv, px + h + lanes])
                plsc.store_scatter(obuf, [chv, dv, h + lanes], v)
        return carry

    lax.fori_loop(0, nch, body, 0)


def _sc_gather(infeat_h, lp_h, lt_h, fda_h, coords_h,
               cls_o, feat_o, pse_o, lab_o,
               cvec, gbuf, obuf):
    w = lax.axis_index("s") * _NC + lax.axis_index("c")

    def jbody(p, carry):
        b, py, px, py8, qy = _patch_coords(coords_h, cvec, p)
        pltpu.sync_copy(
            fda_h.at[b, pl.ds(w * _CCH, _CCH), pl.ds(py8, _NROW), :],
            gbuf)
        _realign(gbuf, obuf, _CCH, qy, px)
        pltpu.sync_copy(obuf, feat_o.at[p, pl.ds(w * _CCH, _CCH)])
        return carry

    lax.fori_loop(0, _NPATCH, jbody, 0)

    @pl.when(w >= 16)
    def _():
        p = w - 16
        b, py, px, py8, qy = _patch_coords(coords_h, cvec, p)
        for src_h, dst_o, nch in ((infeat_h, cls_o, 2), (lp_h, pse_o, 1),
                                  (lt_h, lab_o, 1)):
            pltpu.sync_copy(src_h.at[b, :, pl.ds(py8, _NROW), :],
                            gbuf.at[pl.ds(0, nch)])
            _realign(gbuf.at[pl.ds(0, nch)], obuf.at[pl.ds(0, nch)],
                     nch, qy, px)
            pltpu.sync_copy(obuf.at[pl.ds(0, nch)], dst_o.at[p])


@functools.lru_cache(maxsize=1)
def _make_gather_call():
    return functools.partial(
        pl.kernel,
        out_type=(
            jax.ShapeDtypeStruct((_NPATCH, 2, _KER, _KER), jnp.float32),
            jax.ShapeDtypeStruct((_NPATCH, _CFEAT, _KER, _KER), jnp.float32),
            jax.ShapeDtypeStruct((_NPATCH, 1, _KER, _KER), jnp.float32),
            jax.ShapeDtypeStruct((_NPATCH, 1, _KER, _KER), jnp.float32),
        ),
        mesh=plsc.VectorSubcoreMesh(core_axis_name="c", subcore_axis_name="s"),
        scratch_types=[
            pltpu.VMEM((1, _LANES), jnp.int32),
            pltpu.VMEM((_CCH, _NROW, _ORISIZE), jnp.float32),
            pltpu.VMEM((_CCH, _KER, _KER), jnp.float32),
        ],
        compiler_params=pltpu.CompilerParams(needs_layout_passes=False),
    )(_sc_gather)


def kernel(infeat, labelTpesudo, labelT, FeatureDA, k):
    del k
    prov, pxy, coords = _score_call(infeat)
    cls, feat, pse, lab = _make_gather_call()(
        infeat, labelTpesudo, labelT, FeatureDA, coords)
    return (cls, feat, pse, lab, prov, pxy)


# speedup vs baseline: 10.4890x; 1.1136x over previous
import functools

import jax
import jax.numpy as jnp
from jax import lax
from jax.experimental import pallas as pl
from jax.experimental.pallas import tpu as pltpu
from jax.experimental.pallas import tpu_sc as plsc

_ORISIZE = 256
_KER = 32
_P = _ORISIZE - _KER + 1
_HALF = _KER // 2
_B = 4
_CFEAT = 256
_NPATCH = 16

_NC, _NS = 2, 16


def _score_kernel(infeat_ref, prov_ref, pxy_ref, coords_ref):
    iy = lax.broadcasted_iota(jnp.int32, (_P, _P), 0)
    ix = lax.broadcasted_iota(jnp.int32, (_P, _P), 1)
    flat = iy * _P + ix
    big = jnp.int32(1 << 30)
    for b in range(_B):
        x0 = infeat_ref[b, 0]
        x1 = infeat_ref[b, 1]
        m = jnp.maximum(x0, x1)
        e0 = jnp.exp(x0 - m)
        e1 = jnp.exp(x1 - m)
        den = e0 + e1
        for c in range(2):
            sm = (e0 if c == 0 else e1) / den
            s = sm
            for d in (1, 2, 4, 8, 16):
                s = s + jnp.concatenate(
                    [s[:, d:], jnp.zeros((_ORISIZE, d), s.dtype)], axis=1)
            for d in (1, 2, 4, 8, 16):
                s = s + jnp.concatenate(
                    [s[d:, :], jnp.zeros((d, _ORISIZE), s.dtype)], axis=0)
            pooled = s[:_P, :_P] * (1.0 / (_KER * _KER))

            m1 = jnp.max(pooled)
            idx1 = jnp.min(jnp.where(pooled == m1, flat, big))
            py1 = idx1 // _P
            px1 = idx1 % _P
            oy0 = jnp.maximum(0, py1 - _HALF)
            oy1 = jnp.minimum(_P, py1 + _HALF)
            ox0 = jnp.maximum(0, px1 - _HALF)
            ox1 = jnp.minimum(_P, px1 + _HALF)
            region = (iy >= oy0) & (iy < oy1) & (ix >= ox0) & (ix < ox1)
            filt2 = jnp.where(region, jnp.float32(0.0), pooled)
            m2 = jnp.max(filt2)
            idx2 = jnp.min(jnp.where(filt2 == m2, flat, big))
            py2 = idx2 // _P
            px2 = idx2 % _P

            for kk, (mv, py, px) in enumerate(((m1, py1, px1),
                                               (m2, py2, px2))):
                p = c * (2 * _B) + kk * _B + b
                prov_ref[p] = mv
                pxy_ref[p, 0, 0] = px
                pxy_ref[p, 0, 1] = px + (_KER - 1)
                pxy_ref[p, 1, 0] = py
                pxy_ref[p, 1, 1] = py + (_KER - 1)
                coords_ref[p, 0, 0] = b
                coords_ref[p, 0, 1] = py
                coords_ref[p, 0, 2] = px
                for f in range(3, 16):
                    coords_ref[p, 0, f] = 0


_score_call = pl.pallas_call(
    _score_kernel,
    out_shape=(
        jax.ShapeDtypeStruct((_NPATCH,), jnp.float32),
        jax.ShapeDtypeStruct((_NPATCH, 2, 2), jnp.int32),
        jax.ShapeDtypeStruct((_NPATCH, 1, 16), jnp.int32),
    ),
    in_specs=[pl.BlockSpec(memory_space=pltpu.VMEM)],
    out_specs=(
        pl.BlockSpec(memory_space=pltpu.SMEM),
        pl.BlockSpec(memory_space=pltpu.SMEM),
        pl.BlockSpec(memory_space=pltpu.SMEM),
    ),
)


_NROW = 40
_CCH = 8
_NCHUNK = _CFEAT // _CCH
_LANES = 16


def _patch_coords(coords_h, cvec, p):
    pltpu.sync_copy(coords_h.at[p], cvec)
    vec = cvec[0]
    lanes = lax.iota(jnp.int32, _LANES)
    zero = jnp.zeros((_LANES,), jnp.int32)
    b = jnp.max(jnp.where(lanes == 0, vec, zero))
    py = jnp.max(jnp.where(lanes == 1, vec, zero))
    px = jnp.max(jnp.where(lanes == 2, vec, zero))
    py8 = jnp.minimum((py // 8) * 8, _ORISIZE - _NROW)
    qy = py - py8
    xt = (px // 128) * 128
    pxl = px - xt
    return b, py, px, py8, qy, xt, pxl


def _realign(gbuf, obuf, nch, qy, px):
    lanes = lax.iota(jnp.int32, _LANES)

    def body(ch, carry):
        chv = jnp.full((_LANES,), ch, jnp.int32)
        for r in range(_KER):
            rv = jnp.full((_LANES,), qy + r, jnp.int32)
            dv = jnp.full((_LANES,), r, jnp.int32)
            for h in (0, _LANES):
                v = plsc.load_gather(gbuf, [chv, rv, px + h + lanes])
                plsc.store_scatter(obuf, [chv, dv, h + lanes], v)
        return carry

    lax.fori_loop(0, nch, body, 0)


def _sc_gather(infeat_h, lp_h, lt_h, fda_h, coords_h,
               cls_o, feat_o, pse_o, lab_o,
               cvec, gbuf, obuf):
    w = lax.axis_index("s") * _NC + lax.axis_index("c")

    def _fetch(src, b, ch0, nch, py8, xt, pxl):
        xt0 = pl.multiple_of(xt, 128)
        pltpu.sync_copy(src.at[b, pl.ds(ch0, nch), pl.ds(py8, _NROW),
                               pl.ds(xt0, 128)],
                        gbuf.at[pl.ds(0, nch), :, pl.ds(0, 128)])

        @pl.when(pxl > 128 - _KER)
        def _():
            pltpu.sync_copy(src.at[b, pl.ds(ch0, nch), pl.ds(py8, _NROW),
                                   pl.ds(128, 128)],
                            gbuf.at[pl.ds(0, nch), :, pl.ds(128, 128)])

    def jbody(p, carry):
        b, py, px, py8, qy, xt, pxl = _patch_coords(coords_h, cvec, p)
        _fetch(fda_h, b, w * _CCH, _CCH, py8, xt, pxl)
        _realign(gbuf, obuf, _CCH, qy, pxl)
        pltpu.sync_copy(obuf, feat_o.at[p, pl.ds(w * _CCH, _CCH)])
        return carry

    lax.fori_loop(0, _NPATCH, jbody, 0)

    @pl.when(w >= 16)
    def _():
        p = w - 16
        b, py, px, py8, qy, xt, pxl = _patch_coords(coords_h, cvec, p)
        for src_h, dst_o, nch in ((infeat_h, cls_o, 2), (lp_h, pse_o, 1),
                                  (lt_h, lab_o, 1)):
            _fetch(src_h, b, 0, nch, py8, xt, pxl)
            _realign(gbuf.at[pl.ds(0, nch)], obuf.at[pl.ds(0, nch)],
                     nch, qy, pxl)
            pltpu.sync_copy(obuf.at[pl.ds(0, nch)], dst_o.at[p])


@functools.lru_cache(maxsize=1)
def _make_gather_call():
    return functools.partial(
        pl.kernel,
        out_type=(
            jax.ShapeDtypeStruct((_NPATCH, 2, _KER, _KER), jnp.float32),
            jax.ShapeDtypeStruct((_NPATCH, _CFEAT, _KER, _KER), jnp.float32),
            jax.ShapeDtypeStruct((_NPATCH, 1, _KER, _KER), jnp.float32),
            jax.ShapeDtypeStruct((_NPATCH, 1, _KER, _KER), jnp.float32),
        ),
        mesh=plsc.VectorSubcoreMesh(core_axis_name="c", subcore_axis_name="s"),
        scratch_types=[
            pltpu.VMEM((1, _LANES), jnp.int32),
            pltpu.VMEM((_CCH, _NROW, _ORISIZE), jnp.float32),
            pltpu.VMEM((_CCH, _KER, _KER), jnp.float32),
        ],
        compiler_params=pltpu.CompilerParams(needs_layout_passes=False),
    )(_sc_gather)


def kernel(infeat, labelTpesudo, labelT, FeatureDA, k):
    del k
    prov, pxy, coords = _score_call(infeat)
    cls, feat, pse, lab = _make_gather_call()(
        infeat, labelTpesudo, labelT, FeatureDA, coords)
    return (cls, feat, pse, lab, prov, pxy)


# speedup vs baseline: 13.1651x; 1.2551x over previous
import functools

import jax
import jax.numpy as jnp
from jax import lax
from jax.experimental import pallas as pl
from jax.experimental.pallas import tpu as pltpu
from jax.experimental.pallas import tpu_sc as plsc

_ORISIZE = 256
_KER = 32
_P = _ORISIZE - _KER + 1
_HALF = _KER // 2
_B = 4
_CFEAT = 256
_NPATCH = 16

_NC, _NS = 2, 16


def _argmax2d(val, flat, big):
    m = jnp.max(jnp.max(val, axis=2), axis=1)
    eq = val == m[:, None, None]
    idx = jnp.min(jnp.min(jnp.where(eq, flat[None], big), axis=2), axis=1)
    return m, idx


def _score_kernel(infeat_ref, prov_ref, pxv_ref, pyv_ref):
    x0 = infeat_ref[:, 0]
    x1 = infeat_ref[:, 1]
    m = jnp.maximum(x0, x1)
    e0 = jnp.exp(x0 - m)
    e1 = jnp.exp(x1 - m)
    den = e0 + e1
    s = jnp.concatenate([e0 / den, e1 / den], axis=0)
    for d in (1, 2, 4, 8, 16):
        s = s + jnp.concatenate(
            [s[:, :, d:], jnp.zeros((2 * _B, _ORISIZE, d), s.dtype)], axis=2)
    for d in (1, 2, 4, 8, 16):
        s = s + jnp.concatenate(
            [s[:, d:, :], jnp.zeros((2 * _B, d, _ORISIZE), s.dtype)], axis=1)
    pooled = s[:, :_P, :_P] * (1.0 / (_KER * _KER))

    iy = lax.broadcasted_iota(jnp.int32, (_P, _P), 0)
    ix = lax.broadcasted_iota(jnp.int32, (_P, _P), 1)
    flat = iy * _P + ix
    big = jnp.int32(1 << 30)

    m1, idx1 = _argmax2d(pooled, flat, big)
    py1 = idx1 // _P
    px1 = idx1 % _P
    oy0 = jnp.maximum(0, py1 - _HALF)[:, None, None]
    oy1 = jnp.minimum(_P, py1 + _HALF)[:, None, None]
    ox0 = jnp.maximum(0, px1 - _HALF)[:, None, None]
    ox1 = jnp.minimum(_P, px1 + _HALF)[:, None, None]
    region = ((iy[None] >= oy0) & (iy[None] < oy1)
              & (ix[None] >= ox0) & (ix[None] < ox1))
    filt2 = jnp.where(region, jnp.float32(0.0), pooled)
    m2, idx2 = _argmax2d(filt2, flat, big)
    py2 = idx2 // _P
    px2 = idx2 % _P

    def order(v1, v2):
        return jnp.concatenate(
            [v1[0:_B], v2[0:_B], v1[_B:2 * _B], v2[_B:2 * _B]])

    prov_ref[...] = order(m1, m2)
    pxv_ref[...] = order(px1, px2)
    pyv_ref[...] = order(py1, py2)


_score_call = pl.pallas_call(
    _score_kernel,
    out_shape=(
        jax.ShapeDtypeStruct((_NPATCH,), jnp.float32),
        jax.ShapeDtypeStruct((_NPATCH,), jnp.int32),
        jax.ShapeDtypeStruct((_NPATCH,), jnp.int32),
    ),
    in_specs=[pl.BlockSpec(memory_space=pltpu.VMEM)],
)


_NROW = 40
_CCH = 4
_LANES = 16


def _realign(gbuf, obuf, nch, qy, pxl):
    lanes = lax.iota(jnp.int32, _LANES)

    def body(ch, carry):
        chv = jnp.full((_LANES,), ch, jnp.int32)
        for r in range(_KER):
            rv = jnp.full((_LANES,), qy + r, jnp.int32)
            dv = jnp.full((_LANES,), r, jnp.int32)
            for h in (0, _LANES):
                v = plsc.load_gather(gbuf, [chv, rv, pxl + h + lanes])
                plsc.store_scatter(obuf, [chv, dv, h + lanes], v)
        return carry

    lax.fori_loop(0, nch, body, 0)


def _sc_gather(infeat_h, lp_h, lt_h, fda_h, coords_h,
               cls_o, feat_o, pse_o, lab_o,
               cbuf, gb0, gb1, ob0, ob1, si0, si1, so0, so1):
    w = lax.axis_index("s") * _NC + lax.axis_index("c")
    pltpu.sync_copy(coords_h, cbuf)
    lanes = lax.iota(jnp.int32, _LANES)
    zero = jnp.zeros((_LANES,), jnp.int32)

    def coords_for(p):
        vec = cbuf[p, 0]
        b = jnp.max(jnp.where(lanes == 0, vec, zero))
        py = jnp.max(jnp.where(lanes == 1, vec, zero))
        px = jnp.max(jnp.where(lanes == 2, vec, zero))
        py8 = jnp.minimum((py // 8) * 8, _ORISIZE - _NROW)
        xt = pl.multiple_of((px // 128) * 128, 128)
        pxl = px - (px // 128) * 128
        return b, py - py8, py8, xt, pxl

    def in_copies(gb, sem, j):
        p = j // 2
        ch0 = 8 * w + _CCH * (j % 2)
        b, qy, py8, xt, pxl = coords_for(p)
        c0 = pltpu.make_async_copy(
            fda_h.at[b, pl.ds(ch0, _CCH), pl.ds(py8, _NROW),
                     pl.ds(xt, 128)],
            gb.at[:, :, pl.ds(0, 128)], sem)
        c1 = pltpu.make_async_copy(
            fda_h.at[b, pl.ds(ch0, _CCH), pl.ds(py8, _NROW),
                     pl.ds(128, 128)],
            gb.at[:, :, pl.ds(128, 128)], sem)
        return c0, c1, pxl > 128 - _KER

    def fire_in(gb, sem, j):
        c0, c1, span = in_copies(gb, sem, j)
        c0.start()

        @pl.when(span)
        def _():
            c1.start()

    def wait_in(gb, sem, j):
        c0, c1, span = in_copies(gb, sem, j)
        c0.wait()

        @pl.when(span)
        def _():
            c1.wait()

    def out_copy(ob, sem, j):
        p = j // 2
        ch0 = 8 * w + _CCH * (j % 2)
        return pltpu.make_async_copy(
            ob, feat_o.at[p, pl.ds(ch0, _CCH)], sem)

    fire_in(gb0, si0, 0)

    def body2(i, carry):
        j0 = 2 * i
        j1 = 2 * i + 1
        fire_in(gb1, si1, j1)
        wait_in(gb0, si0, j0)

        @pl.when(i > 0)
        def _():
            out_copy(ob0, so0, j0 - 2).wait()

        b, qy, py8, xt, pxl = coords_for(j0 // 2)
        _realign(gb0, ob0, _CCH, qy, pxl)
        out_copy(ob0, so0, j0).start()

        @pl.when(i < _NPATCH - 1)
        def _():
            fire_in(gb0, si0, j0 + 2)

        wait_in(gb1, si1, j1)

        @pl.when(i > 0)
        def _():
            out_copy(ob1, so1, j1 - 2).wait()

        b, qy, py8, xt, pxl = coords_for(j1 // 2)
        _realign(gb1, ob1, _CCH, qy, pxl)
        out_copy(ob1, so1, j1).start()
        return carry

    lax.fori_loop(0, _NPATCH, body2, 0)
    out_copy(ob0, so0, 2 * _NPATCH - 2).wait()
    out_copy(ob1, so1, 2 * _NPATCH - 1).wait()

    @pl.when(w >= 16)
    def _():
        p = w - 16
        b, qy, py8, xt, pxl = coords_for(p)
        for src_h, dst_o, nch in ((infeat_h, cls_o, 2), (lp_h, pse_o, 1),
                                  (lt_h, lab_o, 1)):
            pltpu.sync_copy(
                src_h.at[b, pl.ds(0, nch), pl.ds(py8, _NROW),
                         pl.ds(xt, 128)],
                gb0.at[pl.ds(0, nch), :, pl.ds(0, 128)])

            @pl.when(pxl > 128 - _KER)
            def _2():
                pltpu.sync_copy(
                    src_h.at[b, pl.ds(0, nch), pl.ds(py8, _NROW),
                             pl.ds(128, 128)],
                    gb0.at[pl.ds(0, nch), :, pl.ds(128, 128)])

            _realign(gb0.at[pl.ds(0, nch)], ob0.at[pl.ds(0, nch)],
                     nch, qy, pxl)
            pltpu.sync_copy(ob0.at[pl.ds(0, nch)], dst_o.at[p])


@functools.lru_cache(maxsize=1)
def _make_gather_call():
    return functools.partial(
        pl.kernel,
        out_type=(
            jax.ShapeDtypeStruct((_NPATCH, 2, _KER, _KER), jnp.float32),
            jax.ShapeDtypeStruct((_NPATCH, _CFEAT, _KER, _KER), jnp.float32),
            jax.ShapeDtypeStruct((_NPATCH, 1, _KER, _KER), jnp.float32),
            jax.ShapeDtypeStruct((_NPATCH, 1, _KER, _KER), jnp.float32),
        ),
        mesh=plsc.VectorSubcoreMesh(core_axis_name="c", subcore_axis_name="s"),
        scratch_types=[
            pltpu.VMEM((_NPATCH, 1, _LANES), jnp.int32),
            pltpu.VMEM((_CCH, _NROW, _ORISIZE), jnp.float32),
            pltpu.VMEM((_CCH, _NROW, _ORISIZE), jnp.float32),
            pltpu.VMEM((_CCH, _KER, _KER), jnp.float32),
            pltpu.VMEM((_CCH, _KER, _KER), jnp.float32),
            pltpu.SemaphoreType.DMA,
            pltpu.SemaphoreType.DMA,
            pltpu.SemaphoreType.DMA,
            pltpu.SemaphoreType.DMA,
        ],
        compiler_params=pltpu.CompilerParams(needs_layout_passes=False),
    )(_sc_gather)


def kernel(infeat, labelTpesudo, labelT, FeatureDA, k):
    del k
    prov, pxv, pyv = _score_call(infeat)
    pointXY = jnp.stack(
        [jnp.stack([pxv, pxv + (_KER - 1)], axis=-1),
         jnp.stack([pyv, pyv + (_KER - 1)], axis=-1)], axis=1)
    bv = (jnp.arange(_NPATCH, dtype=jnp.int32) % _B)
    coords = jnp.concatenate(
        [bv[:, None], pyv[:, None], pxv[:, None],
         jnp.zeros((_NPATCH, 13), jnp.int32)], axis=1)
    coords = coords.reshape(_NPATCH, 1, 16)
    cls, feat, pse, lab = _make_gather_call()(
        infeat, labelTpesudo, labelT, FeatureDA, coords)
    return (cls, feat, pse, lab, prov, pointXY)


# speedup vs baseline: 13.3515x; 1.0142x over previous
import functools

import jax
import jax.numpy as jnp
from jax import lax
from jax.experimental import pallas as pl
from jax.experimental.pallas import tpu as pltpu
from jax.experimental.pallas import tpu_sc as plsc

_ORISIZE = 256
_KER = 32
_P = _ORISIZE - _KER + 1
_HALF = _KER // 2
_B = 4
_CFEAT = 256
_NPATCH = 16

_NC, _NS = 2, 16


def _argmax2d(val, flat, big):
    m = jnp.max(jnp.max(val, axis=2), axis=1)
    eq = val == m[:, None, None]
    idx = jnp.min(jnp.min(jnp.where(eq, flat[None], big), axis=2), axis=1)
    return m, idx


def _score_kernel(infeat_ref, prov_ref, pxy_ref, coords_ref):
    x0 = infeat_ref[:, 0]
    x1 = infeat_ref[:, 1]
    m = jnp.maximum(x0, x1)
    e0 = jnp.exp(x0 - m)
    e1 = jnp.exp(x1 - m)
    den = e0 + e1
    s = jnp.concatenate([e0 / den, e1 / den], axis=0)
    for d in (1, 2, 4, 8, 16):
        s = s + jnp.concatenate(
            [s[:, :, d:], jnp.zeros((2 * _B, _ORISIZE, d), s.dtype)], axis=2)
    for d in (1, 2, 4, 8, 16):
        s = s + jnp.concatenate(
            [s[:, d:, :], jnp.zeros((2 * _B, d, _ORISIZE), s.dtype)], axis=1)
    pooled = s[:, :_P, :_P] * (1.0 / (_KER * _KER))

    iy = lax.broadcasted_iota(jnp.int32, (_P, _P), 0)
    ix = lax.broadcasted_iota(jnp.int32, (_P, _P), 1)
    flat = iy * _P + ix
    big = jnp.int32(1 << 30)

    m1, idx1 = _argmax2d(pooled, flat, big)
    py1 = idx1 // _P
    px1 = idx1 % _P
    oy0 = jnp.maximum(0, py1 - _HALF)[:, None, None]
    oy1 = jnp.minimum(_P, py1 + _HALF)[:, None, None]
    ox0 = jnp.maximum(0, px1 - _HALF)[:, None, None]
    ox1 = jnp.minimum(_P, px1 + _HALF)[:, None, None]
    region = ((iy[None] >= oy0) & (iy[None] < oy1)
              & (ix[None] >= ox0) & (ix[None] < ox1))
    filt2 = jnp.where(region, jnp.float32(0.0), pooled)
    m2, idx2 = _argmax2d(filt2, flat, big)
    py2 = idx2 // _P
    px2 = idx2 % _P

    def order(v1, v2):
        return jnp.concatenate(
            [v1[0:_B], v2[0:_B], v1[_B:2 * _B], v2[_B:2 * _B]])

    prov_ref[...] = order(m1, m2)
    pxv = order(px1, px2)
    pyv = order(py1, py2)

    def bcast(v, shape):
        return lax.broadcast_in_dim(v, shape, (0,))

    a1 = lax.broadcasted_iota(jnp.int32, (_NPATCH, 2, 2), 1)
    a2 = lax.broadcasted_iota(jnp.int32, (_NPATCH, 2, 2), 2)
    px3 = bcast(pxv, (_NPATCH, 2, 2))
    py3 = bcast(pyv, (_NPATCH, 2, 2))
    pxy_ref[...] = (jnp.where(a1 == 0, px3, py3)
                    + jnp.where(a2 == 1, jnp.int32(_KER - 1), jnp.int32(0)))

    col = lax.broadcasted_iota(jnp.int32, (_NPATCH, 1, 16), 2)
    bvc = lax.broadcasted_iota(jnp.int32, (_NPATCH, 1, 16), 0) % _B
    pyc = bcast(pyv, (_NPATCH, 1, 16))
    pxc = bcast(pxv, (_NPATCH, 1, 16))
    coords_ref[...] = jnp.where(
        col == 0, bvc,
        jnp.where(col == 1, pyc,
                  jnp.where(col == 2, pxc, jnp.int32(0))))


_score_call = pl.pallas_call(
    _score_kernel,
    out_shape=(
        jax.ShapeDtypeStruct((_NPATCH,), jnp.float32),
        jax.ShapeDtypeStruct((_NPATCH, 2, 2), jnp.int32),
        jax.ShapeDtypeStruct((_NPATCH, 1, 16), jnp.int32),
    ),
    in_specs=[pl.BlockSpec(memory_space=pltpu.VMEM)],
)


_NROW = 40
_CCH = 4
_LANES = 16


def _realign(gbuf, obuf, nch, qy, pxl):
    lanes = lax.iota(jnp.int32, _LANES)

    def body(ch, carry):
        chv = jnp.full((_LANES,), ch, jnp.int32)
        for r in range(_KER):
            rv = jnp.full((_LANES,), qy + r, jnp.int32)
            dv = jnp.full((_LANES,), r, jnp.int32)
            for h in (0, _LANES):
                v = plsc.load_gather(gbuf, [chv, rv, pxl + h + lanes])
                plsc.store_scatter(obuf, [chv, dv, h + lanes], v)
        return carry

    lax.fori_loop(0, nch, body, 0)


def _sc_gather(infeat_h, lp_h, lt_h, fda_h, coords_h,
               cls_o, feat_o, pse_o, lab_o,
               cbuf, gb0, gb1, ob0, ob1, si0, si1, so0, so1):
    w = lax.axis_index("s") * _NC + lax.axis_index("c")
    pltpu.sync_copy(coords_h, cbuf)
    lanes = lax.iota(jnp.int32, _LANES)
    zero = jnp.zeros((_LANES,), jnp.int32)

    def coords_for(p):
        vec = cbuf[p, 0]
        b = jnp.max(jnp.where(lanes == 0, vec, zero))
        py = jnp.max(jnp.where(lanes == 1, vec, zero))
        px = jnp.max(jnp.where(lanes == 2, vec, zero))
        py8 = jnp.minimum((py // 8) * 8, _ORISIZE - _NROW)
        xt = pl.multiple_of((px // 128) * 128, 128)
        pxl = px - (px // 128) * 128
        return b, py - py8, py8, xt, pxl

    def in_copies(gb, sem, j):
        p = j // 2
        ch0 = 8 * w + _CCH * (j % 2)
        b, qy, py8, xt, pxl = coords_for(p)
        c0 = pltpu.make_async_copy(
            fda_h.at[b, pl.ds(ch0, _CCH), pl.ds(py8, _NROW),
                     pl.ds(xt, 128)],
            gb.at[:, :, pl.ds(0, 128)], sem)
        c1 = pltpu.make_async_copy(
            fda_h.at[b, pl.ds(ch0, _CCH), pl.ds(py8, _NROW),
                     pl.ds(128, 128)],
            gb.at[:, :, pl.ds(128, 128)], sem)
        return c0, c1, pxl > 128 - _KER

    def fire_in(gb, sem, j):
        c0, c1, span = in_copies(gb, sem, j)
        c0.start()

        @pl.when(span)
        def _():
            c1.start()

    def wait_in(gb, sem, j):
        c0, c1, span = in_copies(gb, sem, j)
        c0.wait()

        @pl.when(span)
        def _():
            c1.wait()

    def out_copy(ob, sem, j):
        p = j // 2
        ch0 = 8 * w + _CCH * (j % 2)
        return pltpu.make_async_copy(
            ob, feat_o.at[p, pl.ds(ch0, _CCH)], sem)

    fire_in(gb0, si0, 0)

    def body2(i, carry):
        j0 = 2 * i
        j1 = 2 * i + 1
        fire_in(gb1, si1, j1)
        wait_in(gb0, si0, j0)

        @pl.when(i > 0)
        def _():
            out_copy(ob0, so0, j0 - 2).wait()

        b, qy, py8, xt, pxl = coords_for(j0 // 2)
        _realign(gb0, ob0, _CCH, qy, pxl)
        out_copy(ob0, so0, j0).start()

        @pl.when(i < _NPATCH - 1)
        def _():
            fire_in(gb0, si0, j0 + 2)

        wait_in(gb1, si1, j1)

        @pl.when(i > 0)
        def _():
            out_copy(ob1, so1, j1 - 2).wait()

        b, qy, py8, xt, pxl = coords_for(j1 // 2)
        _realign(gb1, ob1, _CCH, qy, pxl)
        out_copy(ob1, so1, j1).start()
        return carry

    lax.fori_loop(0, _NPATCH, body2, 0)
    out_copy(ob0, so0, 2 * _NPATCH - 2).wait()
    out_copy(ob1, so1, 2 * _NPATCH - 1).wait()

    @pl.when(w >= 16)
    def _():
        p = w - 16
        b, qy, py8, xt, pxl = coords_for(p)
        for src_h, dst_o, nch in ((infeat_h, cls_o, 2), (lp_h, pse_o, 1),
                                  (lt_h, lab_o, 1)):
            pltpu.sync_copy(
                src_h.at[b, pl.ds(0, nch), pl.ds(py8, _NROW),
                         pl.ds(xt, 128)],
                gb0.at[pl.ds(0, nch), :, pl.ds(0, 128)])

            @pl.when(pxl > 128 - _KER)
            def _2():
                pltpu.sync_copy(
                    src_h.at[b, pl.ds(0, nch), pl.ds(py8, _NROW),
                             pl.ds(128, 128)],
                    gb0.at[pl.ds(0, nch), :, pl.ds(128, 128)])

            _realign(gb0.at[pl.ds(0, nch)], ob0.at[pl.ds(0, nch)],
                     nch, qy, pxl)
            pltpu.sync_copy(ob0.at[pl.ds(0, nch)], dst_o.at[p])


@functools.lru_cache(maxsize=1)
def _make_gather_call():
    return functools.partial(
        pl.kernel,
        out_type=(
            jax.ShapeDtypeStruct((_NPATCH, 2, _KER, _KER), jnp.float32),
            jax.ShapeDtypeStruct((_NPATCH, _CFEAT, _KER, _KER), jnp.float32),
            jax.ShapeDtypeStruct((_NPATCH, 1, _KER, _KER), jnp.float32),
            jax.ShapeDtypeStruct((_NPATCH, 1, _KER, _KER), jnp.float32),
        ),
        mesh=plsc.VectorSubcoreMesh(core_axis_name="c", subcore_axis_name="s"),
        scratch_types=[
            pltpu.VMEM((_NPATCH, 1, _LANES), jnp.int32),
            pltpu.VMEM((_CCH, _NROW, _ORISIZE), jnp.float32),
            pltpu.VMEM((_CCH, _NROW, _ORISIZE), jnp.float32),
            pltpu.VMEM((_CCH, _KER, _KER), jnp.float32),
            pltpu.VMEM((_CCH, _KER, _KER), jnp.float32),
            pltpu.SemaphoreType.DMA,
            pltpu.SemaphoreType.DMA,
            pltpu.SemaphoreType.DMA,
            pltpu.SemaphoreType.DMA,
        ],
        compiler_params=pltpu.CompilerParams(needs_layout_passes=False),
    )(_sc_gather)


def kernel(infeat, labelTpesudo, labelT, FeatureDA, k):
    del k
    prov, pointXY, coords = _score_call(infeat)
    cls, feat, pse, lab = _make_gather_call()(
        infeat, labelTpesudo, labelT, FeatureDA, coords)
    return (cls, feat, pse, lab, prov, pointXY)
